# baseline (speedup 1.0000x reference)
import math

import jax
import jax.numpy as jnp
from jax import lax
from jax.experimental import pallas as pl
from jax.experimental.pallas import tpu as pltpu
from jax.experimental.pallas import tpu_sc as plsc

_LANES = 16


def _positional_encoding_2d(seq_len, d):
    pos = jnp.arange(seq_len, dtype=jnp.float32)[:, None]
    even_idx = jnp.arange(0, d, 2, dtype=jnp.float32)
    odd_idx = jnp.arange(1, d, 2, dtype=jnp.float32)
    even_div = jnp.power(10000.0, 2.0 * even_idx / d)
    odd_div = jnp.power(10000.0, 2.0 * odd_idx / d)
    pe = jnp.zeros((seq_len, d), dtype=jnp.float32)
    pe = pe.at[:, 0::2].set(jnp.sin(pos / even_div))
    pe = pe.at[:, 1::2].set(jnp.cos(pos / odd_div))
    return pe


def kernel(x, table):
    b, s = x.shape
    v, d = table.shape
    scale = math.sqrt(d)
    r = b * s

    info = plsc.get_sparse_core_info()
    nc, ns = info.num_cores, info.num_subcores
    nw = nc * ns

    cr = 128
    assert r % (nw * cr) == 0 and d % _LANES == 0
    rpw = r // nw
    nch = rpw // cr
    assert nch % 2 == 0
    groups = d // _LANES

    pe2 = jnp.concatenate([_positional_encoding_2d(s, d)] * 2, axis=0).reshape(-1)
    xr = x.astype(jnp.int32).reshape(nw, nch, cr)

    mesh = plsc.VectorSubcoreMesh(core_axis_name="c", subcore_axis_name="s")

    def body(x_hbm, pe_hbm, table_hbm, out_hbm,
             idx_v, pe_v, gbuf0, gbuf1, wbuf0, wbuf1,
             gsem0, gsem1, wsem0, wsem1):
        wid = lax.axis_index("s") * nc + lax.axis_index("c")
        pltpu.sync_copy(x_hbm.at[wid], idx_v)
        pltpu.sync_copy(pe_hbm, pe_v)
        row_base = wid * rpw

        def issue_gather(c, gbuf, gsem):
            pltpu.async_copy(table_hbm.at[idx_v.at[c]], gbuf, gsem)

        def wait_gather(c, gbuf, gsem):
            pltpu.make_async_copy(table_hbm.at[idx_v.at[c]], gbuf, gsem).wait()

        def out_slice(c):
            return out_hbm.at[pl.ds(row_base + c * cr, cr)]

        issue_gather(0, gbuf0, gsem0)
        issue_gather(1, gbuf1, gsem1)

        @pl.loop(0, nch // 2)
        def _outer(t):
            for par, gbuf, wbuf, gsem, wsem in (
                (0, gbuf0, wbuf0, gsem0, wsem0),
                (1, gbuf1, wbuf1, gsem1, wsem1),
            ):
                c = 2 * t + par
                wait_gather(c, gbuf, gsem)

                @pl.when(t > 0)
                def _drain_prev_write():
                    pltpu.make_async_copy(wbuf, out_slice(c - 2), wsem).wait()

                pe_base = lax.rem(c * cr, s) * d

                @pl.loop(0, cr)
                def _row(rr):
                    roff = pe_base + rr * d
                    for g in range(groups):
                        vec = gbuf[rr, pl.ds(g * _LANES, _LANES)]
                        pvec = pe_v[pl.ds(roff + g * _LANES, _LANES)]
                        wbuf[rr, pl.ds(g * _LANES, _LANES)] = vec * scale + pvec

                pltpu.async_copy(wbuf, out_slice(c), wsem)

                @pl.when(c + 2 < nch)
                def _next_gather():
                    issue_gather(c + 2, gbuf, gsem)

        pltpu.make_async_copy(wbuf0, out_slice(nch - 2), wsem0).wait()
        pltpu.make_async_copy(wbuf1, out_slice(nch - 1), wsem1).wait()

    out = pl.kernel(
        body,
        out_type=jax.ShapeDtypeStruct((r, d), jnp.float32),
        mesh=mesh,
        compiler_params=pltpu.CompilerParams(use_tc_tiling_on_sc=False),
        scratch_types=[
            pltpu.VMEM((nch, cr), jnp.int32),
            pltpu.VMEM((2 * s * d,), jnp.float32),
            pltpu.VMEM((cr, d), jnp.float32),
            pltpu.VMEM((cr, d), jnp.float32),
            pltpu.VMEM((cr, d), jnp.float32),
            pltpu.VMEM((cr, d), jnp.float32),
            pltpu.SemaphoreType.DMA,
            pltpu.SemaphoreType.DMA,
            pltpu.SemaphoreType.DMA,
            pltpu.SemaphoreType.DMA,
        ],
    )(xr, pe2, table)
    return out.reshape(b, s, d)
